# baseline (speedup 1.0000x reference)
import functools

import jax
import jax.numpy as jnp
from jax import lax
from jax.experimental import pallas as pl
from jax.experimental.pallas import tpu as pltpu
from jax.experimental.pallas import tpu_sc as plsc

VOCAB = 100000
MAXLEN = 200
EMBED_DIM = 128
BATCH = 4096

_INFO = plsc.get_sparse_core_info()
_NC = _INFO.num_cores
_NS = _INFO.num_subcores
_NW = _NC * _NS
_ROWS_PER_W = BATCH * MAXLEN // _NW
_CHUNK = MAXLEN
_NCHUNK = _ROWS_PER_W // _CHUNK
_NBUF = 4


def _body(x_hbm, tok_hbm, pos_hbm, out_hbm,
          ix0, ix1, ix2, ix3, pos_v, b0, b1, b2, b3,
          isem0, isem1, isem2, isem3, gsem0, gsem1, gsem2, gsem3,
          osem0, osem1, osem2, osem3):
    wid = lax.axis_index("s") * _NC + lax.axis_index("c")
    base_row = wid * _ROWS_PER_W

    ix = (ix0, ix1, ix2, ix3)
    isem = (isem0, isem1, isem2, isem3)
    buf = (b0, b1, b2, b3)
    gsem = (gsem0, gsem1, gsem2, gsem3)
    osem = (osem0, osem1, osem2, osem3)

    def idx_src(c):
        return x_hbm.at[pl.ds(base_row + c * _CHUNK, _CHUNK)]

    def start_idx(c, s):
        pltpu.async_copy(idx_src(c), ix[s], isem[s])

    def wait_idx(c, s):
        pltpu.make_async_copy(idx_src(c), ix[s], isem[s]).wait()

    def start_gather(s, b):
        pltpu.async_copy(tok_hbm.at[ix[s]], buf[b], gsem[b])

    def wait_gather(s, b):
        pltpu.make_async_copy(tok_hbm.at[ix[s]], buf[b], gsem[b]).wait()

    def add_pos(b):
        def add_row(i, carry):
            for j in range(EMBED_DIM // 16):
                sl = pl.ds(j * 16, 16)
                buf[b][i, sl] += pos_v[i, sl]
            return carry

        lax.fori_loop(0, _CHUNK, add_row, 0)

    def start_store(c, b):
        pltpu.async_copy(
            buf[b], out_hbm.at[pl.ds(base_row + c * _CHUNK, _CHUNK)], osem[b])

    def wait_store(c, b):
        pltpu.make_async_copy(
            buf[b], out_hbm.at[pl.ds(base_row + c * _CHUNK, _CHUNK)],
            osem[b]).wait()

    pltpu.sync_copy(pos_hbm, pos_v)
    pltpu.sync_copy(idx_src(0), ix[0])
    pltpu.sync_copy(idx_src(1), ix[1])
    start_idx(2, 2)
    start_idx(3, 3)
    start_gather(0, 0)
    start_gather(1, 1)

    def step(c, b, first):
        wait_gather(b, b)
        add_pos(b)
        start_store(c, b)

        @pl.when(c + 4 < _NCHUNK)
        def _():
            start_idx(c + 4, b)

        @pl.when(c + 2 < _NCHUNK)
        def _():
            nb = (b + 2) % _NBUF
            wait_idx(c + 2, nb)
            if not first:
                wait_store(c - 2, nb)
            start_gather(nb, nb)

    step(0, 0, True)
    step(1, 1, True)

    def group_step(g, carry):
        for k in range(_NBUF):
            step(2 + 4 * g + k, (2 + k) % _NBUF, False)
        return carry

    lax.fori_loop(0, (_NCHUNK - 4) // _NBUF, group_step, 0)

    step(_NCHUNK - 2, (_NCHUNK - 2) % _NBUF, False)
    step(_NCHUNK - 1, (_NCHUNK - 1) % _NBUF, False)

    for c in range(_NCHUNK - 4, _NCHUNK):
        wait_store(c, c % _NBUF)


@jax.jit
def _run(x_flat, token_table, pos_table):
    k = functools.partial(
        pl.kernel,
        mesh=plsc.VectorSubcoreMesh(core_axis_name="c", subcore_axis_name="s"),
        out_type=jax.ShapeDtypeStruct((BATCH * MAXLEN, EMBED_DIM), jnp.float32),
        scratch_types=[
            pltpu.VMEM((_CHUNK,), jnp.int32),
            pltpu.VMEM((_CHUNK,), jnp.int32),
            pltpu.VMEM((_CHUNK,), jnp.int32),
            pltpu.VMEM((_CHUNK,), jnp.int32),
            pltpu.VMEM((MAXLEN, EMBED_DIM), jnp.float32),
            pltpu.VMEM((_CHUNK, EMBED_DIM), jnp.float32),
            pltpu.VMEM((_CHUNK, EMBED_DIM), jnp.float32),
            pltpu.VMEM((_CHUNK, EMBED_DIM), jnp.float32),
            pltpu.VMEM((_CHUNK, EMBED_DIM), jnp.float32),
            pltpu.SemaphoreType.DMA,
            pltpu.SemaphoreType.DMA,
            pltpu.SemaphoreType.DMA,
            pltpu.SemaphoreType.DMA,
            pltpu.SemaphoreType.DMA,
            pltpu.SemaphoreType.DMA,
            pltpu.SemaphoreType.DMA,
            pltpu.SemaphoreType.DMA,
            pltpu.SemaphoreType.DMA,
            pltpu.SemaphoreType.DMA,
            pltpu.SemaphoreType.DMA,
            pltpu.SemaphoreType.DMA,
        ],
    )(_body)
    return k(x_flat, token_table, pos_table)


def kernel(x, token_table, pos_table):
    x_flat = x.astype(jnp.int32).reshape(-1)
    out = _run(x_flat, token_table, pos_table)
    return out.reshape(BATCH, MAXLEN, EMBED_DIM)
